# baseline (speedup 1.0000x reference)
import functools
import math

import jax
import jax.numpy as jnp
from jax import lax
from jax.experimental import pallas as pl
from jax.experimental.pallas import tpu as pltpu
from jax.experimental.pallas import tpu_sc as plsc

_NC = 2
_NS = 16
_NW = _NC * _NS
_CHUNK = 128
_CD = 256
_LCM = math.lcm(2 * _CHUNK, 2 * _CD)
_BLK = 1024


def _sc_mesh():
    return plsc.VectorSubcoreMesh(core_axis_name="c", subcore_axis_name="s")


def _sc_degree(rowp, ones_d, zeros_d, np_, ept, d):
    stripe = np_ // _NS
    nch = ept // _CD

    @functools.partial(
        pl.kernel,
        out_type=jax.ShapeDtypeStruct((_NC * np_, d), jnp.float32),
        mesh=_sc_mesh(),
        scratch_types=[
            pltpu.VMEM((_CD,), jnp.int32),
            pltpu.VMEM((_CD,), jnp.int32),
            pltpu.VMEM((_CD, d), jnp.float32),
        ] + [pltpu.SemaphoreType.DMA] * 2 + [
            pltpu.VMEM_SHARED((np_, d), jnp.float32),
        ],
    )
    def k(row_hbm, ones_hbm, zeros_hbm, out_hbm, r0, r1, ones_v, *rest):
        sr0, sr1 = rest[:2]
        agg_sh = rest[2]
        cid = lax.axis_index("c")
        sid = lax.axis_index("s")
        wid = sid * _NC + cid
        pltpu.sync_copy(ones_hbm, ones_v)
        pltpu.sync_copy(zeros_hbm.at[pl.ds(sid * stripe, stripe)],
                        agg_sh.at[pl.ds(sid * stripe, stripe)])
        pltpu.async_copy(row_hbm.at[pl.ds(wid * ept, _CD)], r0, sr0)
        pltpu.async_copy(row_hbm.at[pl.ds(wid * ept + _CD, _CD)], r1, sr1)
        plsc.subcore_barrier()

        @pl.loop(0, nch, step=2)
        def _(j):
            pltpu.make_async_copy(row_hbm.at[pl.ds(wid * ept, _CD)], r0,
                                  sr0).wait()
            pltpu.sync_copy(ones_v, agg_sh.at[r0], add=True)

            @pl.when(j + 2 < nch)
            def _():
                pltpu.async_copy(row_hbm.at[pl.ds(wid * ept + (j + 2) * _CD, _CD)],
                                 r0, sr0)

            pltpu.make_async_copy(row_hbm.at[pl.ds(wid * ept, _CD)], r1,
                                  sr1).wait()
            pltpu.sync_copy(ones_v, agg_sh.at[r1], add=True)

            @pl.when(j + 3 < nch)
            def _():
                pltpu.async_copy(row_hbm.at[pl.ds(wid * ept + (j + 3) * _CD, _CD)],
                                 r1, sr1)

        plsc.subcore_barrier()
        pltpu.sync_copy(agg_sh.at[pl.ds(sid * stripe, stripe)],
                        out_hbm.at[pl.ds(cid * np_ + sid * stripe, stripe)])

    return k(rowp, ones_d, zeros_d).reshape(_NC, np_, d)


def _sc_spmm(rowp3, colp3, x, zeros_d, np_, nch, d):
    stripe = np_ // _NS

    @functools.partial(
        pl.kernel,
        out_type=jax.ShapeDtypeStruct((_NC * np_, d), jnp.float32),
        mesh=_sc_mesh(),
        scratch_types=[
            pltpu.VMEM((nch, _CHUNK), jnp.int32),
            pltpu.VMEM((nch, _CHUNK), jnp.int32),
            pltpu.VMEM((_CHUNK, d), jnp.float32),
            pltpu.VMEM_SHARED((np_, d), jnp.float32),
        ],
    )
    def k(row_hbm, col_hbm, x_hbm, zeros_hbm, out_hbm,
          row_v, col_v, rows_v, agg_sh):
        cid = lax.axis_index("c")
        sid = lax.axis_index("s")
        wid = sid * _NC + cid
        pltpu.sync_copy(row_hbm.at[wid], row_v)
        pltpu.sync_copy(col_hbm.at[wid], col_v)
        pltpu.sync_copy(zeros_hbm.at[pl.ds(sid * stripe, stripe)],
                        agg_sh.at[pl.ds(sid * stripe, stripe)])
        plsc.subcore_barrier()

        @pl.loop(0, nch)
        def _(j):
            pltpu.sync_copy(x_hbm.at[col_v.at[j]], rows_v)
            pltpu.sync_copy(rows_v, agg_sh.at[row_v.at[j]], add=True)

        plsc.subcore_barrier()
        pltpu.sync_copy(agg_sh.at[pl.ds(sid * stripe, stripe)],
                        out_hbm.at[pl.ds(cid * np_ + sid * stripe, stripe)])

    return k(rowp3, colp3, x, zeros_d).reshape(_NC, np_, d)


def _tc_prep(degp, p0, np_, d):
    def body(degp_ref, p0_ref, s_ref, x_ref):
        deg = degp_ref[0, :, 0:1] + degp_ref[1, :, 0:1]
        s = lax.rsqrt(deg + 1e-10)
        sb = jnp.broadcast_to(s, p0_ref.shape)
        s_ref[...] = sb
        x_ref[...] = p0_ref[...] * sb

    grid = (np_ // _BLK,)
    return pl.pallas_call(
        body,
        grid=grid,
        in_specs=[
            pl.BlockSpec((2, _BLK, d), lambda i: (0, i, 0)),
            pl.BlockSpec((_BLK, d), lambda i: (i, 0)),
        ],
        out_specs=[
            pl.BlockSpec((_BLK, d), lambda i: (i, 0)),
            pl.BlockSpec((_BLK, d), lambda i: (i, 0)),
        ],
        out_shape=[
            jax.ShapeDtypeStruct((np_, d), jnp.float32),
            jax.ShapeDtypeStruct((np_, d), jnp.float32),
        ],
    )(degp, p0)


def _tc_layer(partials, s, w, acc, scale, np_, d):
    def body(p_ref, s_ref, w_ref, acc_ref, accout_ref, x_ref):
        u = p_ref[0] + p_ref[1]
        m = jnp.dot(u, w_ref[...], preferred_element_type=jnp.float32)
        sv = s_ref[...]
        t = jnp.maximum(sv * m, 0.0)
        accout_ref[...] = (acc_ref[...] + t) * scale
        x_ref[...] = sv * t

    grid = (np_ // _BLK,)
    return pl.pallas_call(
        body,
        grid=grid,
        in_specs=[
            pl.BlockSpec((2, _BLK, d), lambda i: (0, i, 0)),
            pl.BlockSpec((_BLK, d), lambda i: (i, 0)),
            pl.BlockSpec((d, d), lambda i: (0, 0)),
            pl.BlockSpec((_BLK, d), lambda i: (i, 0)),
        ],
        out_specs=[
            pl.BlockSpec((_BLK, d), lambda i: (i, 0)),
            pl.BlockSpec((_BLK, d), lambda i: (i, 0)),
        ],
        out_shape=[
            jax.ShapeDtypeStruct((np_, d), jnp.float32),
            jax.ShapeDtypeStruct((np_, d), jnp.float32),
        ],
    )(partials, s, w, acc)


def kernel(edge_index, user_embeds, item_embeds, W):
    nu = user_embeds.shape[0]
    n = nu + item_embeds.shape[0]
    d = user_embeds.shape[1]
    e = edge_index.shape[1]
    nl = W.shape[0]

    ept = math.ceil(e / (_NW * _LCM)) * _LCM
    ep = ept * _NW
    np_ = math.ceil((n + 1) / _BLK) * _BLK

    row = edge_index[0]
    col = edge_index[1]
    pad = jnp.full((ep - e,), n, dtype=jnp.int32)
    rowp = jnp.concatenate([row, pad])
    colp = jnp.concatenate([col, pad])

    p0 = jnp.concatenate([user_embeds, item_embeds], axis=0)
    p0 = jnp.pad(p0, ((0, np_ - n), (0, 0)))

    zeros_d = jnp.zeros((np_, d), jnp.float32)
    ones_d = jnp.ones((_CD, d), jnp.float32)

    degp = _sc_degree(rowp, ones_d, zeros_d, np_, ept, d)
    s, x = _tc_prep(degp, p0, np_, d)

    nch = ept // _CHUNK
    rowp3 = rowp.reshape(_NW, nch, _CHUNK)
    colp3 = colp.reshape(_NW, nch, _CHUNK)

    acc = p0
    for layer in range(nl):
        partials = _sc_spmm(rowp3, colp3, x, zeros_d, np_, nch, d)
        scale = 1.0 / (nl + 1) if layer == nl - 1 else 1.0
        acc, x = _tc_layer(partials, s, W[layer], acc, scale, np_, d)

    return acc[:nu], acc[nu:n]


# speedup vs baseline: 1.1387x; 1.1387x over previous
import functools
import math

import jax
import jax.numpy as jnp
from jax import lax
from jax.experimental import pallas as pl
from jax.experimental.pallas import tpu as pltpu
from jax.experimental.pallas import tpu_sc as plsc

_NC = 2
_NS = 16
_NW = _NC * _NS
_CHUNK = 128
_CD = 256
_LCM = math.lcm(2 * _CHUNK, 2 * _CD)
_BLK = 1024


def _sc_mesh():
    return plsc.VectorSubcoreMesh(core_axis_name="c", subcore_axis_name="s")


def _sc_degree(rowp3, ones_d, zeros_d, np_, nch, d):
    stripe = np_ // _NS

    @functools.partial(
        pl.kernel,
        out_type=jax.ShapeDtypeStruct((_NC * np_, d), jnp.float32),
        mesh=_sc_mesh(),
        scratch_types=[
            pltpu.VMEM((nch, _CHUNK), jnp.int32),
            pltpu.VMEM((_CHUNK, d), jnp.float32),
            pltpu.VMEM_SHARED((np_, d), jnp.float32),
        ],
    )
    def k(row_hbm, ones_hbm, zeros_hbm, out_hbm, idx_v, ones_v, agg_sh):
        cid = lax.axis_index("c")
        sid = lax.axis_index("s")
        wid = sid * _NC + cid
        pltpu.sync_copy(row_hbm.at[wid], idx_v)
        pltpu.sync_copy(ones_hbm, ones_v)
        pltpu.sync_copy(zeros_hbm.at[pl.ds(sid * stripe, stripe)],
                        agg_sh.at[pl.ds(sid * stripe, stripe)])
        plsc.subcore_barrier()

        @pl.loop(0, nch)
        def _(j):
            pltpu.sync_copy(ones_v, agg_sh.at[idx_v.at[j]], add=True)

        plsc.subcore_barrier()
        pltpu.sync_copy(agg_sh.at[pl.ds(sid * stripe, stripe)],
                        out_hbm.at[pl.ds(cid * np_ + sid * stripe, stripe)])

    return k(rowp3, ones_d, zeros_d).reshape(_NC, np_, d)


def _sc_spmm(rowp3, colp3, x, zeros_d, np_, nch, d):
    stripe = np_ // _NS

    @functools.partial(
        pl.kernel,
        out_type=jax.ShapeDtypeStruct((_NC * np_, d), jnp.float32),
        mesh=_sc_mesh(),
        scratch_types=[
            pltpu.VMEM((nch, _CHUNK), jnp.int32),
            pltpu.VMEM((nch, _CHUNK), jnp.int32),
            pltpu.VMEM((_CHUNK, d), jnp.float32),
            pltpu.VMEM_SHARED((np_, d), jnp.float32),
        ],
    )
    def k(row_hbm, col_hbm, x_hbm, zeros_hbm, out_hbm,
          row_v, col_v, rows_v, agg_sh):
        cid = lax.axis_index("c")
        sid = lax.axis_index("s")
        wid = sid * _NC + cid
        pltpu.sync_copy(row_hbm.at[wid], row_v)
        pltpu.sync_copy(col_hbm.at[wid], col_v)
        pltpu.sync_copy(zeros_hbm.at[pl.ds(sid * stripe, stripe)],
                        agg_sh.at[pl.ds(sid * stripe, stripe)])
        plsc.subcore_barrier()

        @pl.loop(0, nch)
        def _(j):
            pltpu.sync_copy(x_hbm.at[col_v.at[j]], rows_v)
            pltpu.sync_copy(rows_v, agg_sh.at[row_v.at[j]], add=True)

        plsc.subcore_barrier()
        pltpu.sync_copy(agg_sh.at[pl.ds(sid * stripe, stripe)],
                        out_hbm.at[pl.ds(cid * np_ + sid * stripe, stripe)])

    return k(rowp3, colp3, x, zeros_d).reshape(_NC, np_, d)


def _tc_prep(degp, p0, np_, d):
    def body(degp_ref, p0_ref, s_ref, x_ref):
        deg = degp_ref[0, :, 0:1] + degp_ref[1, :, 0:1]
        s = lax.rsqrt(deg + 1e-10)
        sb = jnp.broadcast_to(s, p0_ref.shape)
        s_ref[...] = sb
        x_ref[...] = p0_ref[...] * sb

    grid = (np_ // _BLK,)
    return pl.pallas_call(
        body,
        grid=grid,
        in_specs=[
            pl.BlockSpec((2, _BLK, d), lambda i: (0, i, 0)),
            pl.BlockSpec((_BLK, d), lambda i: (i, 0)),
        ],
        out_specs=[
            pl.BlockSpec((_BLK, d), lambda i: (i, 0)),
            pl.BlockSpec((_BLK, d), lambda i: (i, 0)),
        ],
        out_shape=[
            jax.ShapeDtypeStruct((np_, d), jnp.float32),
            jax.ShapeDtypeStruct((np_, d), jnp.float32),
        ],
    )(degp, p0)


def _tc_layer(partials, s, w, acc, scale, np_, d):
    def body(p_ref, s_ref, w_ref, acc_ref, accout_ref, x_ref):
        u = p_ref[0] + p_ref[1]
        m = jnp.dot(u, w_ref[...], preferred_element_type=jnp.float32)
        sv = s_ref[...]
        t = jnp.maximum(sv * m, 0.0)
        accout_ref[...] = (acc_ref[...] + t) * scale
        x_ref[...] = sv * t

    grid = (np_ // _BLK,)
    return pl.pallas_call(
        body,
        grid=grid,
        in_specs=[
            pl.BlockSpec((2, _BLK, d), lambda i: (0, i, 0)),
            pl.BlockSpec((_BLK, d), lambda i: (i, 0)),
            pl.BlockSpec((d, d), lambda i: (0, 0)),
            pl.BlockSpec((_BLK, d), lambda i: (i, 0)),
        ],
        out_specs=[
            pl.BlockSpec((_BLK, d), lambda i: (i, 0)),
            pl.BlockSpec((_BLK, d), lambda i: (i, 0)),
        ],
        out_shape=[
            jax.ShapeDtypeStruct((np_, d), jnp.float32),
            jax.ShapeDtypeStruct((np_, d), jnp.float32),
        ],
    )(partials, s, w, acc)


def kernel(edge_index, user_embeds, item_embeds, W):
    nu = user_embeds.shape[0]
    n = nu + item_embeds.shape[0]
    d = user_embeds.shape[1]
    e = edge_index.shape[1]
    nl = W.shape[0]

    ept = math.ceil(e / (_NW * _LCM)) * _LCM
    ep = ept * _NW
    np_ = math.ceil((n + 1) / _BLK) * _BLK

    row = edge_index[0]
    col = edge_index[1]
    pad = jnp.full((ep - e,), n, dtype=jnp.int32)
    rowp = jnp.concatenate([row, pad])
    colp = jnp.concatenate([col, pad])

    p0 = jnp.concatenate([user_embeds, item_embeds], axis=0)
    p0 = jnp.pad(p0, ((0, np_ - n), (0, 0)))

    nch = ept // _CHUNK
    rowp3 = rowp.reshape(_NW, nch, _CHUNK)
    colp3 = colp.reshape(_NW, nch, _CHUNK)

    zeros_d = jnp.zeros((np_, d), jnp.float32)
    ones_d = jnp.ones((_CHUNK, d), jnp.float32)

    degp = _sc_degree(rowp3, ones_d, zeros_d, np_, nch, d)
    s, x = _tc_prep(degp, p0, np_, d)

    acc = p0
    for layer in range(nl):
        partials = _sc_spmm(rowp3, colp3, x, zeros_d, np_, nch, d)
        scale = 1.0 / (nl + 1) if layer == nl - 1 else 1.0
        acc, x = _tc_layer(partials, s, W[layer], acc, scale, np_, d)

    return acc[:nu], acc[nu:n]


# speedup vs baseline: 2.6921x; 2.3642x over previous
import functools
import math

import jax
import jax.numpy as jnp
from jax import lax
from jax.experimental import pallas as pl
from jax.experimental.pallas import tpu as pltpu
from jax.experimental.pallas import tpu_sc as plsc

_NC = 2
_NS = 16
_NW = _NC * _NS
_CHUNK = 128
_CD = 256
_LCM = math.lcm(2 * _CHUNK, 2 * _CD)
_BLK = 1024


def _sc_mesh():
    return plsc.VectorSubcoreMesh(core_axis_name="c", subcore_axis_name="s")


def _sc_degree(rowp3, ones_d, zeros_d, np_, nch, d):
    stripe = np_ // _NS

    @functools.partial(
        pl.kernel,
        out_type=jax.ShapeDtypeStruct((_NC * np_, d), jnp.float32),
        mesh=_sc_mesh(),
        scratch_types=[
            pltpu.VMEM((nch, _CHUNK), jnp.int32),
            pltpu.VMEM((_CHUNK, d), jnp.float32),
            pltpu.VMEM_SHARED((np_, d), jnp.float32),
        ],
    )
    def k(row_hbm, ones_hbm, zeros_hbm, out_hbm, idx_v, ones_v, agg_sh):
        cid = lax.axis_index("c")
        sid = lax.axis_index("s")
        wid = sid * _NC + cid
        pltpu.sync_copy(row_hbm.at[wid], idx_v)
        pltpu.sync_copy(ones_hbm, ones_v)
        pltpu.sync_copy(zeros_hbm.at[pl.ds(sid * stripe, stripe)],
                        agg_sh.at[pl.ds(sid * stripe, stripe)])
        plsc.subcore_barrier()

        @pl.loop(0, nch)
        def _(j):
            pltpu.sync_copy(ones_v, agg_sh.at[idx_v.at[j]], add=True)

        plsc.subcore_barrier()
        pltpu.sync_copy(agg_sh.at[pl.ds(sid * stripe, stripe)],
                        out_hbm.at[pl.ds(cid * np_ + sid * stripe, stripe)])

    return k(rowp3, ones_d, zeros_d).reshape(_NC, np_, d)


def _sc_spmm(rowp3, colp3, x, zeros_d, np_, nch, d):
    stripe = np_ // _NS

    @functools.partial(
        pl.kernel,
        out_type=jax.ShapeDtypeStruct((_NC * np_, d), jnp.float32),
        mesh=_sc_mesh(),
        scratch_types=[
            pltpu.VMEM((nch, _CHUNK), jnp.int32),
            pltpu.VMEM((nch, _CHUNK), jnp.int32),
            pltpu.VMEM((_CHUNK, d), jnp.float32),
            pltpu.VMEM_SHARED((np_, d), jnp.float32),
        ],
    )
    def k(row_hbm, col_hbm, x_hbm, zeros_hbm, out_hbm,
          row_v, col_v, rows_v, agg_sh):
        cid = lax.axis_index("c")
        sid = lax.axis_index("s")
        wid = sid * _NC + cid
        pltpu.sync_copy(row_hbm.at[wid], row_v)
        pltpu.sync_copy(col_hbm.at[wid], col_v)
        pltpu.sync_copy(zeros_hbm.at[pl.ds(sid * stripe, stripe)],
                        agg_sh.at[pl.ds(sid * stripe, stripe)])
        plsc.subcore_barrier()

        @pl.loop(0, nch)
        def _(j):
            pltpu.sync_copy(x_hbm.at[col_v.at[j]], rows_v)
            pltpu.sync_copy(rows_v, agg_sh.at[row_v.at[j]], add=True)

        plsc.subcore_barrier()
        pltpu.sync_copy(agg_sh.at[pl.ds(sid * stripe, stripe)],
                        out_hbm.at[pl.ds(cid * np_ + sid * stripe, stripe)])

    return k(rowp3, colp3, x, zeros_d).reshape(_NC, np_, d)


def _tc_prep(degp, p0, np_, d):
    def body(degp_ref, p0_ref, s_ref, x_ref):
        deg = degp_ref[0, :, 0:1] + degp_ref[1, :, 0:1]
        s = lax.rsqrt(deg + 1e-10)
        sb = jnp.broadcast_to(s, p0_ref.shape)
        s_ref[...] = sb
        x_ref[...] = p0_ref[...] * sb

    grid = (np_ // _BLK,)
    return pl.pallas_call(
        body,
        grid=grid,
        in_specs=[
            pl.BlockSpec((2, _BLK, d), lambda i: (0, i, 0)),
            pl.BlockSpec((_BLK, d), lambda i: (i, 0)),
        ],
        out_specs=[
            pl.BlockSpec((_BLK, d), lambda i: (i, 0)),
            pl.BlockSpec((_BLK, d), lambda i: (i, 0)),
        ],
        out_shape=[
            jax.ShapeDtypeStruct((np_, d), jnp.float32),
            jax.ShapeDtypeStruct((np_, d), jnp.float32),
        ],
    )(degp, p0)


def _tc_layer(partials, s, w, acc, scale, np_, d):
    def body(p_ref, s_ref, w_ref, acc_ref, accout_ref, x_ref):
        u = p_ref[0] + p_ref[1]
        m = jnp.dot(u, w_ref[...], preferred_element_type=jnp.float32)
        sv = s_ref[...]
        t = jnp.maximum(sv * m, 0.0)
        accout_ref[...] = (acc_ref[...] + t) * scale
        x_ref[...] = sv * t

    grid = (np_ // _BLK,)
    return pl.pallas_call(
        body,
        grid=grid,
        in_specs=[
            pl.BlockSpec((2, _BLK, d), lambda i: (0, i, 0)),
            pl.BlockSpec((_BLK, d), lambda i: (i, 0)),
            pl.BlockSpec((d, d), lambda i: (0, 0)),
            pl.BlockSpec((_BLK, d), lambda i: (i, 0)),
        ],
        out_specs=[
            pl.BlockSpec((_BLK, d), lambda i: (i, 0)),
            pl.BlockSpec((_BLK, d), lambda i: (i, 0)),
        ],
        out_shape=[
            jax.ShapeDtypeStruct((np_, d), jnp.float32),
            jax.ShapeDtypeStruct((np_, d), jnp.float32),
        ],
    )(partials, s, w, acc)


def kernel(edge_index, user_embeds, item_embeds, W):
    nu = user_embeds.shape[0]
    n = nu + item_embeds.shape[0]
    d = user_embeds.shape[1]
    e = edge_index.shape[1]
    nl = W.shape[0]

    ept = math.ceil(e / (_NW * _LCM)) * _LCM
    ep = ept * _NW
    np_ = math.ceil((n + 1) / _BLK) * _BLK

    row = edge_index[0]
    col = edge_index[1]
    pad = n + jnp.arange(ep - e, dtype=jnp.int32) % (np_ - n)
    rowp = jnp.concatenate([row, pad])
    colp = jnp.concatenate([col, pad])

    p0 = jnp.concatenate([user_embeds, item_embeds], axis=0)
    p0 = jnp.pad(p0, ((0, np_ - n), (0, 0)))

    nch = ept // _CHUNK
    rowp3 = rowp.reshape(_NW, nch, _CHUNK)
    colp3 = colp.reshape(_NW, nch, _CHUNK)

    zeros_d = jnp.zeros((np_, d), jnp.float32)
    ones_d = jnp.ones((_CHUNK, d), jnp.float32)

    degp = _sc_degree(rowp3, ones_d, zeros_d, np_, nch, d)
    s, x = _tc_prep(degp, p0, np_, d)

    acc = p0
    for layer in range(nl):
        partials = _sc_spmm(rowp3, colp3, x, zeros_d, np_, nch, d)
        scale = 1.0 / (nl + 1) if layer == nl - 1 else 1.0
        acc, x = _tc_layer(partials, s, W[layer], acc, scale, np_, d)

    return acc[:nu], acc[nu:n]


# speedup vs baseline: 3.6632x; 1.3607x over previous
import functools
import math

import jax
import jax.numpy as jnp
from jax import lax
from jax.experimental import pallas as pl
from jax.experimental.pallas import tpu as pltpu
from jax.experimental.pallas import tpu_sc as plsc

_NC = 2
_NS = 16
_NW = _NC * _NS
_CHUNK = 128
_CG = 160
_LCM = math.lcm(2 * _CG, _CHUNK)
_BLK = 1024


def _sc_mesh():
    return plsc.VectorSubcoreMesh(core_axis_name="c", subcore_axis_name="s")


def _sc_degree(rowp3, ones_d, zeros_d, np_, nch, d):
    stripe = np_ // _NS

    @functools.partial(
        pl.kernel,
        out_type=jax.ShapeDtypeStruct((_NC * np_, d), jnp.float32),
        mesh=_sc_mesh(),
        scratch_types=[
            pltpu.VMEM((nch, _CHUNK), jnp.int32),
            pltpu.VMEM((_CHUNK, d), jnp.float32),
            pltpu.VMEM_SHARED((np_, d), jnp.float32),
        ],
    )
    def k(row_hbm, ones_hbm, zeros_hbm, out_hbm, idx_v, ones_v, agg_sh):
        cid = lax.axis_index("c")
        sid = lax.axis_index("s")
        wid = sid * _NC + cid
        pltpu.sync_copy(row_hbm.at[wid], idx_v)
        pltpu.sync_copy(ones_hbm, ones_v)
        pltpu.sync_copy(zeros_hbm.at[pl.ds(sid * stripe, stripe)],
                        agg_sh.at[pl.ds(sid * stripe, stripe)])
        plsc.subcore_barrier()

        @pl.loop(0, nch)
        def _(j):
            pltpu.sync_copy(ones_v, agg_sh.at[idx_v.at[j]], add=True)

        plsc.subcore_barrier()
        pltpu.sync_copy(agg_sh.at[pl.ds(sid * stripe, stripe)],
                        out_hbm.at[pl.ds(cid * np_ + sid * stripe, stripe)])

    return k(rowp3, ones_d, zeros_d).reshape(_NC, np_, d)


def _sc_spmm(rowp, colp, x, zeros_d, np_, ept, d):
    stripe = np_ // _NS
    nch = ept // _CG

    @functools.partial(
        pl.kernel,
        out_type=jax.ShapeDtypeStruct((_NC * np_, d), jnp.float32),
        mesh=_sc_mesh(),
        scratch_types=[
            pltpu.VMEM((_CG,), jnp.int32), pltpu.VMEM((_CG,), jnp.int32),
            pltpu.VMEM((_CG,), jnp.int32), pltpu.VMEM((_CG,), jnp.int32),
            pltpu.VMEM((2, _CG, d), jnp.float32),
        ] + [pltpu.SemaphoreType.DMA] * 6 + [
            pltpu.VMEM_SHARED((np_, d), jnp.float32),
        ],
    )
    def k(row_hbm, col_hbm, x_hbm, zeros_hbm, out_hbm,
          c0, r0, c1, r1, rows_v, *rest):
        sc0, sr0, sc1, sr1, sg0, sg1 = rest[:6]
        agg_sh = rest[6]
        cid = lax.axis_index("c")
        sid = lax.axis_index("s")
        wid = sid * _NC + cid
        pltpu.sync_copy(zeros_hbm.at[pl.ds(sid * stripe, stripe)],
                        agg_sh.at[pl.ds(sid * stripe, stripe)])
        pltpu.async_copy(col_hbm.at[pl.ds(wid * ept, _CG)], c0, sc0)
        pltpu.async_copy(row_hbm.at[pl.ds(wid * ept, _CG)], r0, sr0)
        pltpu.async_copy(col_hbm.at[pl.ds(wid * ept + _CG, _CG)], c1, sc1)
        pltpu.async_copy(row_hbm.at[pl.ds(wid * ept + _CG, _CG)], r1, sr1)
        pltpu.make_async_copy(col_hbm.at[pl.ds(wid * ept, _CG)], c0,
                              sc0).wait()
        pltpu.async_copy(x_hbm.at[c0], rows_v.at[0], sg0)
        pltpu.make_async_copy(col_hbm.at[pl.ds(wid * ept, _CG)], c1,
                              sc1).wait()
        pltpu.async_copy(x_hbm.at[c1], rows_v.at[1], sg1)
        plsc.subcore_barrier()

        @pl.loop(0, nch, step=2)
        def _(j):
            pltpu.make_async_copy(x_hbm.at[c0], rows_v.at[0], sg0).wait()
            pltpu.make_async_copy(row_hbm.at[pl.ds(wid * ept, _CG)], r0,
                                  sr0).wait()
            pltpu.sync_copy(rows_v.at[0], agg_sh.at[r0], add=True)

            @pl.when(j + 2 < nch)
            def _():
                pltpu.async_copy(
                    col_hbm.at[pl.ds(wid * ept + (j + 2) * _CG, _CG)],
                    c0, sc0)
                pltpu.async_copy(
                    row_hbm.at[pl.ds(wid * ept + (j + 2) * _CG, _CG)],
                    r0, sr0)
                pltpu.make_async_copy(
                    col_hbm.at[pl.ds(wid * ept, _CG)], c0, sc0).wait()
                pltpu.async_copy(x_hbm.at[c0], rows_v.at[0], sg0)

            pltpu.make_async_copy(x_hbm.at[c1], rows_v.at[1], sg1).wait()
            pltpu.make_async_copy(row_hbm.at[pl.ds(wid * ept, _CG)], r1,
                                  sr1).wait()
            pltpu.sync_copy(rows_v.at[1], agg_sh.at[r1], add=True)

            @pl.when(j + 3 < nch)
            def _():
                pltpu.async_copy(
                    col_hbm.at[pl.ds(wid * ept + (j + 3) * _CG, _CG)],
                    c1, sc1)
                pltpu.async_copy(
                    row_hbm.at[pl.ds(wid * ept + (j + 3) * _CG, _CG)],
                    r1, sr1)
                pltpu.make_async_copy(
                    col_hbm.at[pl.ds(wid * ept, _CG)], c1, sc1).wait()
                pltpu.async_copy(x_hbm.at[c1], rows_v.at[1], sg1)

        plsc.subcore_barrier()
        pltpu.sync_copy(agg_sh.at[pl.ds(sid * stripe, stripe)],
                        out_hbm.at[pl.ds(cid * np_ + sid * stripe, stripe)])

    return k(rowp, colp, x, zeros_d).reshape(_NC, np_, d)


def _tc_prep(degp, p0, np_, d):
    def body(degp_ref, p0_ref, s_ref, x_ref):
        deg = degp_ref[0, :, 0:1] + degp_ref[1, :, 0:1]
        s = lax.rsqrt(deg + 1e-10)
        sb = jnp.broadcast_to(s, p0_ref.shape)
        s_ref[...] = sb
        x_ref[...] = p0_ref[...] * sb

    grid = (np_ // _BLK,)
    return pl.pallas_call(
        body,
        grid=grid,
        in_specs=[
            pl.BlockSpec((2, _BLK, d), lambda i: (0, i, 0)),
            pl.BlockSpec((_BLK, d), lambda i: (i, 0)),
        ],
        out_specs=[
            pl.BlockSpec((_BLK, d), lambda i: (i, 0)),
            pl.BlockSpec((_BLK, d), lambda i: (i, 0)),
        ],
        out_shape=[
            jax.ShapeDtypeStruct((np_, d), jnp.float32),
            jax.ShapeDtypeStruct((np_, d), jnp.float32),
        ],
    )(degp, p0)


def _tc_layer(partials, s, w, acc, scale, np_, d):
    def body(p_ref, s_ref, w_ref, acc_ref, accout_ref, x_ref):
        u = p_ref[0] + p_ref[1]
        m = jnp.dot(u, w_ref[...], preferred_element_type=jnp.float32)
        sv = s_ref[...]
        t = jnp.maximum(sv * m, 0.0)
        accout_ref[...] = (acc_ref[...] + t) * scale
        x_ref[...] = sv * t

    grid = (np_ // _BLK,)
    return pl.pallas_call(
        body,
        grid=grid,
        in_specs=[
            pl.BlockSpec((2, _BLK, d), lambda i: (0, i, 0)),
            pl.BlockSpec((_BLK, d), lambda i: (i, 0)),
            pl.BlockSpec((d, d), lambda i: (0, 0)),
            pl.BlockSpec((_BLK, d), lambda i: (i, 0)),
        ],
        out_specs=[
            pl.BlockSpec((_BLK, d), lambda i: (i, 0)),
            pl.BlockSpec((_BLK, d), lambda i: (i, 0)),
        ],
        out_shape=[
            jax.ShapeDtypeStruct((np_, d), jnp.float32),
            jax.ShapeDtypeStruct((np_, d), jnp.float32),
        ],
    )(partials, s, w, acc)


def kernel(edge_index, user_embeds, item_embeds, W):
    nu = user_embeds.shape[0]
    n = nu + item_embeds.shape[0]
    d = user_embeds.shape[1]
    e = edge_index.shape[1]
    nl = W.shape[0]

    ept = math.ceil(e / (_NW * _LCM)) * _LCM
    ep = ept * _NW
    np_ = math.ceil((n + 1) / _BLK) * _BLK

    row = edge_index[0]
    col = edge_index[1]
    pad = n + jnp.arange(ep - e, dtype=jnp.int32) % (np_ - n)
    rowp = jnp.concatenate([row, pad])
    colp = jnp.concatenate([col, pad])

    p0 = jnp.concatenate([user_embeds, item_embeds], axis=0)
    p0 = jnp.pad(p0, ((0, np_ - n), (0, 0)))

    nch = ept // _CHUNK
    rowp3 = rowp.reshape(_NW, nch, _CHUNK)

    zeros_d = jnp.zeros((np_, d), jnp.float32)
    ones_d = jnp.ones((_CHUNK, d), jnp.float32)

    degp = _sc_degree(rowp3, ones_d, zeros_d, np_, nch, d)
    s, x = _tc_prep(degp, p0, np_, d)

    acc = p0
    for layer in range(nl):
        partials = _sc_spmm(rowp, colp, x, zeros_d, np_, ept, d)
        scale = 1.0 / (nl + 1) if layer == nl - 1 else 1.0
        acc, x = _tc_layer(partials, s, W[layer], acc, scale, np_, d)

    return acc[:nu], acc[nu:n]
